# baseline (speedup 1.0000x reference)
import functools

import jax
import jax.numpy as jnp
from jax import lax
from jax.experimental import pallas as pl
from jax.experimental.pallas import tpu as pltpu
from jax.experimental.pallas import tpu_sc as plsc

E = 128
C = 6
L = 16
NUM_CORES = 2
NUM_SUBCORES = 16
NW = NUM_CORES * NUM_SUBCORES
CB = 64
OT = 128


def _build_sc_call(B):
    b_per_w = B // NW
    n_chunks = b_per_w // CB

    mesh = plsc.VectorSubcoreMesh(
        core_axis_name="c", subcore_axis_name="s",
        num_cores=NUM_CORES, num_subcores=NUM_SUBCORES)

    buf_types = [
        pltpu.VMEM((CB, E), jnp.float32),
        pltpu.VMEM((C * CB, E), jnp.float32),
        pltpu.SemaphoreType.DMA,
    ]
    out_buf_types = [
        pltpu.VMEM((C, OT), jnp.float32),
        pltpu.SemaphoreType.DMA,
    ]
    cpo = OT // CB

    @functools.partial(
        pl.kernel,
        out_type=jax.ShapeDtypeStruct((C, B), jnp.float32),
        mesh=mesh,
        scratch_types=buf_types + buf_types + out_buf_types + out_buf_types + [
            pltpu.VMEM((b_per_w,), jnp.int32),
            pltpu.VMEM((C, b_per_w), jnp.int32),
        ],
    )
    def sc_call(tgt_hbm, ctxT_hbm, ttab_hbm, ctab_hbm, outT_hbm,
                wrows0, crows0, sem0,
                wrows1, crows1, sem1,
                outvT0, semo0, outvT1, semo1,
                tidx, cidx):
        wid = lax.axis_index("s") * NUM_CORES + lax.axis_index("c")
        base = wid * b_per_w
        bufs = [(wrows0, crows0, sem0), (wrows1, crows1, sem1)]
        obufs = [(outvT0, semo0), (outvT1, semo1)]

        head = min(OT, b_per_w)
        pltpu.sync_copy(tgt_hbm.at[pl.ds(base, head)], tidx.at[pl.ds(0, head)])
        pltpu.sync_copy(ctxT_hbm.at[:, pl.ds(base, head)],
                        cidx.at[:, pl.ds(0, head)])

        def issue(ch, buf):
            wrows, crows, sem = buf
            cps = [pltpu.async_copy(
                ttab_hbm.at[tidx.at[pl.ds(ch * CB, CB)]], wrows, sem)]
            for c in range(C):
                cps.append(pltpu.async_copy(
                    ctab_hbm.at[cidx.at[c, pl.ds(ch * CB, CB)]],
                    crows.at[pl.ds(c * CB, CB)], sem))
            return cps

        lane = lax.iota(jnp.int32, L)
        xors = {sh: (lane ^ sh).reshape(L, 1) for sh in (8, 4, 2, 1)}
        dnums = lax.GatherDimensionNumbers(
            offset_dims=(), collapsed_slice_dims=(0,), start_index_map=(0,))

        def swap(v, sh):
            return lax.gather(
                v, xors[sh], dnums, slice_sizes=(1,),
                mode=lax.GatherScatterMode.PROMISE_IN_BOUNDS)

        def compute(ch, buf, outvT):
            wrows, crows, _ = buf
            half = (ch % cpo) * CB

            def grp_body(g, part):
                blk = g // (C * 2)
                rem = g % (C * 2)
                c = rem // 2
                h = rem % 2
                b0 = blk * L + h * (L // 2)
                st = []
                for bb in range(L // 2):
                    wv = [wrows[b0 + bb, pl.ds(L * j, L)]
                          for j in range(E // L)]
                    ps = [wv[j] * crows[c * CB + b0 + bb, pl.ds(L * j, L)]
                          for j in range(E // L)]
                    while len(ps) > 1:
                        ps = [ps[k] + ps[k + 1]
                              for k in range(0, len(ps), 2)]
                    p, lvl = ps[0], 0
                    while st and st[-1][0] == lvl:
                        _, a = st.pop()
                        sh = 1 << lvl
                        u = a + swap(a, sh)
                        w = p + swap(p, sh)
                        p = jnp.where((lane & sh) == 0, u, w)
                        lvl += 1
                    st.append((lvl, p))
                q = st[-1][1]
                u = part + swap(part, 8)
                w = q + swap(q, 8)
                full = jnp.where((lane & 8) == 0, u, w)

                @pl.when(h == 1)
                def _():
                    outvT[c, pl.ds(half + blk * L, L)] = full

                return q

            zero = jnp.zeros((L,), jnp.float32)
            lax.fori_loop(0, (CB // L) * C * 2, grp_body, zero)

        pend_gather = {0: issue(0, bufs[0])}
        if n_chunks > 1:
            pend_gather[1] = issue(1, bufs[1])
        if b_per_w > head:
            pltpu.sync_copy(tgt_hbm.at[pl.ds(base + head, b_per_w - head)],
                            tidx.at[pl.ds(head, b_per_w - head)])
            pltpu.sync_copy(ctxT_hbm.at[:, pl.ds(base + head, b_per_w - head)],
                            cidx.at[:, pl.ds(head, b_per_w - head)])
        pend_out = {}
        for ch in range(n_chunks):
            buf = bufs[ch % 2]
            tile = ch // cpo
            outvT, semo = obufs[tile % 2]
            for cp in pend_gather.pop(ch):
                cp.wait()
            if ch % cpo == 0 and tile - 2 in pend_out:
                pend_out.pop(tile - 2).wait()
            compute(ch, buf, outvT)
            if ch % cpo == cpo - 1:
                b0 = base + tile * OT
                pend_out[tile] = pltpu.async_copy(
                    outvT, outT_hbm.at[:, pl.ds(b0, OT)], semo)
            if ch + 2 < n_chunks:
                pend_gather[ch + 2] = issue(ch + 2, buf)
        for cp in pend_out.values():
            cp.wait()

    return sc_call


def kernel(target, context, target_table, context_table):
    if target.ndim == 2:
        target = jnp.squeeze(target, axis=1)
    B = target.shape[0]
    tgt = target.astype(jnp.int32)
    ctxT = context.astype(jnp.int32).T
    outT = _build_sc_call(B)(tgt, ctxT, target_table, context_table)
    return outT.T


# speedup vs baseline: 1.3844x; 1.3844x over previous
import functools

import jax
import jax.numpy as jnp
from jax import lax
from jax.experimental import pallas as pl
from jax.experimental.pallas import tpu as pltpu
from jax.experimental.pallas import tpu_sc as plsc

E = 128
C = 6
L = 16
NUM_CORES = 2
NUM_SUBCORES = 16
NW = NUM_CORES * NUM_SUBCORES
CB = 32
OT = 128


def _build_sc_call(B):
    b_per_w = B // NW
    n_chunks = b_per_w // CB

    mesh = plsc.VectorSubcoreMesh(
        core_axis_name="c", subcore_axis_name="s",
        num_cores=NUM_CORES, num_subcores=NUM_SUBCORES)

    buf_types = [
        pltpu.VMEM((CB, E), jnp.float32),
        pltpu.VMEM((C * CB, E), jnp.float32),
        pltpu.SemaphoreType.DMA,
    ]
    out_buf_types = [
        pltpu.VMEM((C, OT), jnp.float32),
        pltpu.SemaphoreType.DMA,
    ]
    cpo = OT // CB

    @functools.partial(
        pl.kernel,
        out_type=jax.ShapeDtypeStruct((C, B), jnp.float32),
        mesh=mesh,
        scratch_types=buf_types + buf_types + out_buf_types + out_buf_types + [
            pltpu.VMEM((b_per_w,), jnp.int32),
            pltpu.VMEM((C, b_per_w), jnp.int32),
        ],
    )
    def sc_call(tgt_hbm, ctxT_hbm, ttab_hbm, ctab_hbm, outT_hbm,
                wrows0, crows0, sem0,
                wrows1, crows1, sem1,
                outvT0, semo0, outvT1, semo1,
                tidx, cidx):
        wid = lax.axis_index("s") * NUM_CORES + lax.axis_index("c")
        base = wid * b_per_w
        bufs = [(wrows0, crows0, sem0), (wrows1, crows1, sem1)]
        obufs = [(outvT0, semo0), (outvT1, semo1)]

        head = min(OT, b_per_w)
        pltpu.sync_copy(tgt_hbm.at[pl.ds(base, head)], tidx.at[pl.ds(0, head)])
        pltpu.sync_copy(ctxT_hbm.at[:, pl.ds(base, head)],
                        cidx.at[:, pl.ds(0, head)])

        def issue(ch, buf):
            wrows, crows, sem = buf
            cps = [pltpu.async_copy(
                ttab_hbm.at[tidx.at[pl.ds(ch * CB, CB)]], wrows, sem)]
            for c in range(C):
                cps.append(pltpu.async_copy(
                    ctab_hbm.at[cidx.at[c, pl.ds(ch * CB, CB)]],
                    crows.at[pl.ds(c * CB, CB)], sem))
            return cps

        lane = lax.iota(jnp.int32, L)
        xors = {sh: (lane ^ sh).reshape(L, 1) for sh in (8, 4, 2, 1)}
        dnums = lax.GatherDimensionNumbers(
            offset_dims=(), collapsed_slice_dims=(0,), start_index_map=(0,))

        def swap(v, sh):
            return lax.gather(
                v, xors[sh], dnums, slice_sizes=(1,),
                mode=lax.GatherScatterMode.PROMISE_IN_BOUNDS)

        def compute(ch, buf, outvT):
            wrows, crows, _ = buf
            half = (ch % cpo) * CB

            def grp_body(g, parts):
                blk = g // ((C // 2) * 2)
                rem = g % ((C // 2) * 2)
                cp = rem // 2
                h = rem % 2
                b0 = blk * L + h * (L // 2)
                sts = [[], []]
                for bb in range(L // 2):
                    wv = [wrows[b0 + bb, pl.ds(L * j, L)]
                          for j in range(E // L)]
                    for i in range(2):
                        c = cp * 2 + i
                        ps = [wv[j] * crows[c * CB + b0 + bb, pl.ds(L * j, L)]
                              for j in range(E // L)]
                        while len(ps) > 1:
                            ps = [ps[k] + ps[k + 1]
                                  for k in range(0, len(ps), 2)]
                        p, lvl, st = ps[0], 0, sts[i]
                        while st and st[-1][0] == lvl:
                            _, a = st.pop()
                            sh = 1 << lvl
                            u = a + swap(a, sh)
                            w = p + swap(p, sh)
                            p = jnp.where((lane & sh) == 0, u, w)
                            lvl += 1
                        st.append((lvl, p))
                fulls = []
                for i in range(2):
                    q = sts[i][-1][1]
                    u = parts[i] + swap(parts[i], 8)
                    w = q + swap(q, 8)
                    fulls.append((jnp.where((lane & 8) == 0, u, w), q))

                @pl.when(h == 1)
                def _():
                    for i in range(2):
                        outvT[cp * 2 + i, pl.ds(half + blk * L, L)] = fulls[i][0]

                return (fulls[0][1], fulls[1][1])

            zero = jnp.zeros((L,), jnp.float32)
            lax.fori_loop(0, (CB // L) * (C // 2) * 2, grp_body, (zero, zero))

        pend_gather = {0: issue(0, bufs[0])}
        if n_chunks > 1:
            pend_gather[1] = issue(1, bufs[1])
        if b_per_w > head:
            pltpu.sync_copy(tgt_hbm.at[pl.ds(base + head, b_per_w - head)],
                            tidx.at[pl.ds(head, b_per_w - head)])
            pltpu.sync_copy(ctxT_hbm.at[:, pl.ds(base + head, b_per_w - head)],
                            cidx.at[:, pl.ds(head, b_per_w - head)])
        pend_out = {}
        for ch in range(n_chunks):
            buf = bufs[ch % 2]
            tile = ch // cpo
            outvT, semo = obufs[tile % 2]
            for cp in pend_gather.pop(ch):
                cp.wait()
            if ch % cpo == 0 and tile - 2 in pend_out:
                pend_out.pop(tile - 2).wait()
            compute(ch, buf, outvT)
            if ch % cpo == cpo - 1:
                b0 = base + tile * OT
                pend_out[tile] = pltpu.async_copy(
                    outvT, outT_hbm.at[:, pl.ds(b0, OT)], semo)
            if ch + 2 < n_chunks:
                pend_gather[ch + 2] = issue(ch + 2, buf)
        for cp in pend_out.values():
            cp.wait()

    return sc_call


def kernel(target, context, target_table, context_table):
    if target.ndim == 2:
        target = jnp.squeeze(target, axis=1)
    B = target.shape[0]
    tgt = target.astype(jnp.int32)
    ctxT = context.astype(jnp.int32).T
    outT = _build_sc_call(B)(tgt, ctxT, target_table, context_table)
    return outT.T
